# baseline (speedup 1.0000x reference)
import jax
import jax.numpy as jnp
from jax.experimental import pallas as pl
from jax.experimental.pallas import tpu as pltpu

_LOG2E = 1.4426950408889634
_BM = 512
_VE = 128


def _gat_body(h_ref, wv_ref, wq_ref, wk_ref, a_ref, b_ref, o_ref, k_s, ve_s):
    i = pl.program_id(0)
    n_heads = wq_ref.shape[0]
    out_dim = wv_ref.shape[2]
    att = wq_ref.shape[2]
    c = jnp.float32(_LOG2E / float(att) ** 0.5)

    @pl.when(i >= 0)
    def _():
        hm = h_ref[...]
        for hh in range(n_heads):
            k_s[hh] = jnp.dot(
                hm, wk_ref[hh], preferred_element_type=jnp.float32
            ).astype(jnp.bfloat16)
            ve_s[hh, :, :out_dim] = jnp.dot(
                hm, wv_ref[hh], preferred_element_type=jnp.float32
            ).astype(jnp.bfloat16)
            ve_s[hh, :, out_dim:] = jnp.ones(
                (hm.shape[0], _VE - out_dim), jnp.bfloat16)

    hb = h_ref[pl.ds(i * _BM, _BM), :]
    am = a_ref[...].astype(jnp.bfloat16)
    for hh in range(n_heads):
        q = (jnp.dot(hb, wq_ref[hh], preferred_element_type=jnp.float32)
             * c).astype(jnp.bfloat16)
        s = jax.lax.dot_general(q, k_s[hh], (((1,), (1,)), ((), ())),
                                preferred_element_type=jnp.float32)
        l = jnp.maximum(s, 0.2 * s)
        e = jnp.exp2(l).astype(jnp.bfloat16) * am
        oe = jax.lax.dot_general(e, ve_s[hh], (((1,), (0,)), ((), ())),
                                 preferred_element_type=jnp.float32)
        denom = oe[:, out_dim:out_dim + 1] + 1e-37
        o = oe[:, :out_dim] / denom + b_ref[hh]
        o_ref[:, hh * out_dim:(hh + 1) * out_dim] = jnp.where(
            o > 0, o, jnp.exp(jnp.minimum(o, 0.0)) - 1.0)


def kernel(h, a, kernel, attention_kernel, attention_kernel_2, bias):
    B, N, D = h.shape
    H, _, OUT = kernel.shape
    ATT = attention_kernel.shape[2]
    h2 = h.reshape(N, D)
    a2 = a.reshape(N, N)
    b2 = bias.reshape(H, 1, OUT)
    NB = N // _BM

    out = pl.pallas_call(
        _gat_body,
        grid=(NB,),
        compiler_params=pltpu.CompilerParams(
            dimension_semantics=("parallel",)),
        in_specs=[
            pl.BlockSpec((N, D), lambda i: (0, 0)),
            pl.BlockSpec((H, D, OUT), lambda i: (0, 0, 0)),
            pl.BlockSpec((H, D, ATT), lambda i: (0, 0, 0)),
            pl.BlockSpec((H, D, ATT), lambda i: (0, 0, 0)),
            pl.BlockSpec((_BM, N), lambda i: (i, 0)),
            pl.BlockSpec((H, 1, OUT), lambda i: (0, 0, 0)),
        ],
        out_specs=pl.BlockSpec((_BM, H * OUT), lambda i: (i, 0)),
        out_shape=jax.ShapeDtypeStruct((N, H * OUT), jnp.float32),
        scratch_shapes=[
            pltpu.VMEM((H, N, ATT), jnp.bfloat16),
            pltpu.VMEM((H, N, _VE), jnp.bfloat16),
        ],
    )(h2, kernel, attention_kernel, attention_kernel_2, a2, b2)

    return out.reshape(1, N, H * OUT)


# speedup vs baseline: 1.1132x; 1.1132x over previous
import jax
import jax.numpy as jnp
from jax.experimental import pallas as pl
from jax.experimental.pallas import tpu as pltpu

_LOG2E = 1.4426950408889634
_BM = 512
_VE = 128


def _gat_body(h_ref, wv_ref, wq_ref, wk_ref, a_ref, b_ref, o_ref, k_s, ve_s):
    i = pl.program_id(0)
    n_heads = wq_ref.shape[0]
    out_dim = wv_ref.shape[2]
    att = wq_ref.shape[2]
    c = jnp.float32(_LOG2E / float(att) ** 0.5)

    @pl.when(i == 0)
    def _():
        hm = h_ref[...]
        for hh in range(n_heads):
            k_s[hh] = jnp.dot(hm, wk_ref[hh], preferred_element_type=jnp.float32)
            ve_s[hh, :, :out_dim] = jnp.dot(
                hm, wv_ref[hh], preferred_element_type=jnp.float32)
            ve_s[hh, :, out_dim:] = jnp.ones(
                (hm.shape[0], _VE - out_dim), jnp.float32)

    hb = h_ref[pl.ds(i * _BM, _BM), :]
    am = a_ref[...]
    for hh in range(n_heads):
        q = jnp.dot(hb, wq_ref[hh], preferred_element_type=jnp.float32) * c
        s = jax.lax.dot_general(q, k_s[hh], (((1,), (1,)), ((), ())),
                                preferred_element_type=jnp.float32)
        l = jnp.maximum(s, 0.2 * s)
        e = jnp.exp2(l) * am
        oe = jax.lax.dot_general(e, ve_s[hh], (((1,), (0,)), ((), ())),
                                 preferred_element_type=jnp.float32)
        denom = oe[:, out_dim:out_dim + 1] + 1e-37
        o = oe[:, :out_dim] / denom + b_ref[hh]
        o_ref[:, hh * out_dim:(hh + 1) * out_dim] = jnp.where(
            o > 0, o, jnp.exp(jnp.minimum(o, 0.0)) - 1.0)


def kernel(h, a, kernel, attention_kernel, attention_kernel_2, bias):
    B, N, D = h.shape
    H, _, OUT = kernel.shape
    ATT = attention_kernel.shape[2]
    h2 = h.reshape(N, D)
    a2 = a.reshape(N, N)
    b2 = bias.reshape(H, 1, OUT)
    NB = N // _BM

    out = pl.pallas_call(
        _gat_body,
        grid=(NB,),
        in_specs=[
            pl.BlockSpec((N, D), lambda i: (0, 0)),
            pl.BlockSpec((H, D, OUT), lambda i: (0, 0, 0)),
            pl.BlockSpec((H, D, ATT), lambda i: (0, 0, 0)),
            pl.BlockSpec((H, D, ATT), lambda i: (0, 0, 0)),
            pl.BlockSpec((_BM, N), lambda i: (i, 0)),
            pl.BlockSpec((H, 1, OUT), lambda i: (0, 0, 0)),
        ],
        out_specs=pl.BlockSpec((_BM, H * OUT), lambda i: (i, 0)),
        out_shape=jax.ShapeDtypeStruct((N, H * OUT), jnp.float32),
        scratch_shapes=[
            pltpu.VMEM((H, N, ATT), jnp.float32),
            pltpu.VMEM((H, N, _VE), jnp.float32),
        ],
    )(h2, kernel, attention_kernel, attention_kernel_2, a2, b2)

    return out.reshape(1, N, H * OUT)


# speedup vs baseline: 1.1248x; 1.0104x over previous
import jax
import jax.numpy as jnp
from jax.experimental import pallas as pl
from jax.experimental.pallas import tpu as pltpu

_LOG2E = 1.4426950408889634
_BM = 512


def _gat_body(h_ref, wv_ref, wq_ref, wk_ref, a_ref, b_ref, o_ref, k_s, v_s):
    i = pl.program_id(0)
    n_heads = wq_ref.shape[0]
    out_dim = wv_ref.shape[2]
    att = wq_ref.shape[2]
    c = jnp.float32(_LOG2E / float(att) ** 0.5)

    @pl.when(i == 0)
    def _():
        hm = h_ref[...]
        for hh in range(n_heads):
            k_s[hh] = jnp.dot(hm, wk_ref[hh], preferred_element_type=jnp.float32)
            v_s[hh] = jnp.dot(hm, wv_ref[hh], preferred_element_type=jnp.float32)

    hb = h_ref[pl.ds(i * _BM, _BM), :]
    am = a_ref[...]
    for hh in range(n_heads):
        q = jnp.dot(hb, wq_ref[hh], preferred_element_type=jnp.float32) * c
        s = jax.lax.dot_general(q, k_s[hh], (((1,), (1,)), ((), ())),
                                preferred_element_type=jnp.float32)
        l = jnp.maximum(s, 0.2 * s)
        e = jnp.exp2(l) * am
        denom = jnp.sum(e, axis=1, keepdims=True) + 1e-37
        o = jax.lax.dot_general(e, v_s[hh], (((1,), (0,)), ((), ())),
                                preferred_element_type=jnp.float32)
        o = o / denom + b_ref[hh]
        o_ref[:, hh * out_dim:(hh + 1) * out_dim] = jnp.where(
            o > 0, o, jnp.exp(jnp.minimum(o, 0.0)) - 1.0)


def kernel(h, a, kernel, attention_kernel, attention_kernel_2, bias):
    B, N, D = h.shape
    H, _, OUT = kernel.shape
    ATT = attention_kernel.shape[2]
    h2 = h.reshape(N, D)
    a2 = a.reshape(N, N)
    b2 = bias.reshape(H, 1, OUT)
    NB = N // _BM

    out = pl.pallas_call(
        _gat_body,
        grid=(NB,),
        in_specs=[
            pl.BlockSpec((N, D), lambda i: (0, 0)),
            pl.BlockSpec((H, D, OUT), lambda i: (0, 0, 0)),
            pl.BlockSpec((H, D, ATT), lambda i: (0, 0, 0)),
            pl.BlockSpec((H, D, ATT), lambda i: (0, 0, 0)),
            pl.BlockSpec((_BM, N), lambda i: (i, 0)),
            pl.BlockSpec((H, 1, OUT), lambda i: (0, 0, 0)),
        ],
        out_specs=pl.BlockSpec((_BM, H * OUT), lambda i: (i, 0)),
        out_shape=jax.ShapeDtypeStruct((N, H * OUT), jnp.float32),
        scratch_shapes=[
            pltpu.VMEM((H, N, ATT), jnp.float32),
            pltpu.VMEM((H, N, OUT), jnp.float32),
        ],
    )(h2, kernel, attention_kernel, attention_kernel_2, a2, b2)

    return out.reshape(1, N, H * OUT)
